# baseline (speedup 1.0000x reference)
import jax
import jax.numpy as jnp
from jax import lax
from jax.experimental import pallas as pl
from jax.experimental.pallas import tpu as pltpu
from jax.experimental.pallas import tpu_sc as plsc

N = 10000
H = 128
HH = 64
NE = 4
L = 3

NLANES = 16
NC = 2
NS = 16
NW = NC * NS

NPAD = 10240
ROWS_PW = NPAD // NW
EGRP = 80
NEG = ROWS_PW // EGRP

GRP = 128
NG = 158
EPS = NG * GRP
EPAD = NS * EPS
ZROWS = NPAD // NS

EBLK = 4096
NEBLK = EPAD // EBLK

_mesh = plsc.VectorSubcoreMesh(core_axis_name="c", subcore_axis_name="s")


def _emb_body(emb_hbm, z_hbm, out_hbm, z_v, rows_v, sem):
    c = lax.axis_index("c")
    s = lax.axis_index("s")
    wid = s * NC + c
    base = wid * ROWS_PW
    pltpu.sync_copy(z_hbm.at[pl.ds(base, ROWS_PW)], z_v)

    def body(g, carry):
        pltpu.async_copy(emb_hbm.at[z_v.at[pl.ds(g * EGRP, EGRP)]], rows_v, sem).wait()
        pltpu.sync_copy(rows_v, out_hbm.at[pl.ds(base + g * EGRP, EGRP)])
        return carry

    lax.fori_loop(0, NEG, body, 0)


def _emb_gather(emb, z_pad):
    return pl.kernel(
        _emb_body,
        out_type=jax.ShapeDtypeStruct((NPAD, H), jnp.float32),
        mesh=_mesh,
        scratch_types=[
            pltpu.VMEM((ROWS_PW,), jnp.int32),
            pltpu.VMEM((EGRP, H), jnp.float32),
            pltpu.SemaphoreType.DMA,
        ],
    )(emb, z_pad)


def _eproj_body(ea_ref, w_ref, b_ref, *out_refs):
    e = jnp.dot(ea_ref[...], w_ref[...],
                preferred_element_type=jnp.float32) + b_ref[...]
    for i in range(L * NC):
        out_refs[i][...] = e[:, i * H:(i + 1) * H]


def _eproj(ea_pairs, w2cat, b2cat):
    return pl.pallas_call(
        _eproj_body,
        grid=(NEBLK,),
        in_specs=[
            pl.BlockSpec((EBLK // 2, 2 * NE), lambda i: (i, 0)),
            pl.BlockSpec((2 * NE, L * NC * H), lambda i: (0, 0)),
            pl.BlockSpec((1, L * NC * H), lambda i: (0, 0)),
        ],
        out_specs=[pl.BlockSpec((EBLK // 2, H), lambda i: (i, 0))] * (L * NC),
        out_shape=[jax.ShapeDtypeStruct((EPAD // 2, H), jnp.float32)] * (L * NC),
    )(ea_pairs, w2cat, b2cat)


def _edge_body(h0_hbm, h1_hbm, src_hbm, dst_hbm, e0_hbm, e1_hbm, zero_hbm,
               out_hbm, src_v, dst_v, e_v, rows_v, sem_r, sem_e, sem_sc, aggr_s):
    c = lax.axis_index("c")
    s = lax.axis_index("s")
    ebase = s * EPS

    pltpu.sync_copy(src_hbm.at[pl.ds(ebase, EPS)], src_v)
    pltpu.sync_copy(dst_hbm.at[s], dst_v)

    def issue_rows(g, slot):
        idx = src_v.at[pl.ds(g * GRP, GRP)]

        @pl.when(c == 0)
        def _():
            pltpu.async_copy(h0_hbm.at[idx], rows_v.at[slot], sem_r.at[slot])

        @pl.when(c == 1)
        def _():
            pltpu.async_copy(h1_hbm.at[idx], rows_v.at[slot], sem_r.at[slot])

    def issue_e(g, slot):
        sl = pl.ds((ebase + g * GRP) // 2, GRP // 2)

        @pl.when(c == 0)
        def _():
            pltpu.async_copy(e0_hbm.at[sl], e_v.at[slot], sem_e.at[slot])

        @pl.when(c == 1)
        def _():
            pltpu.async_copy(e1_hbm.at[sl], e_v.at[slot], sem_e.at[slot])

    issue_rows(0, 0)
    issue_e(0, 0)

    pltpu.sync_copy(zero_hbm, aggr_s.at[pl.ds(s * ZROWS, ZROWS)])
    plsc.subcore_barrier()

    def group(g, carry):
        slot = lax.rem(g, 2)
        nslot = 1 - slot
        pltpu.make_async_copy(h0_hbm.at[src_v.at[pl.ds(g * GRP, GRP)]],
                              rows_v.at[slot], sem_r.at[slot]).wait()
        pltpu.make_async_copy(e0_hbm.at[pl.ds((ebase + g * GRP) // 2, GRP // 2)],
                              e_v.at[slot], sem_e.at[slot]).wait()

        @pl.when(g >= 1)
        def _():
            pltpu.make_async_copy(rows_v.at[nslot], aggr_s.at[dst_v.at[g]],
                                  sem_sc.at[nslot]).wait()

        @pl.when(g + 1 < NG)
        def _():
            issue_rows(g + 1, nslot)
            issue_e(g + 1, nslot)

        def block(b, bcarry):
            for q in range(4):
                r = []
                for u in range(4):
                    i = b * NLANES + q * 4 + u
                    erow = b * 8 + (q * 4 + u) // 2
                    ecol = (u % 2) * HH
                    for j in range(HH // NLANES):
                        rv = rows_v[slot, i, pl.ds(j * NLANES, NLANES)]
                        ev = e_v[slot, erow, pl.ds(ecol + j * NLANES, NLANES)]
                        r.append(jnp.maximum(rv + ev, 0.0))
                idx = 0
                for u in range(4):
                    i = b * NLANES + q * 4 + u
                    for j in range(HH // NLANES):
                        rows_v[slot, i, pl.ds(j * NLANES, NLANES)] = r[idx]
                        idx += 1
            return bcarry

        lax.fori_loop(0, GRP // NLANES, block, 0)

        pltpu.async_copy(rows_v.at[slot], aggr_s.at[dst_v.at[g]],
                         sem_sc.at[slot], add=True)
        return carry

    lax.fori_loop(0, NG, group, 0)

    pltpu.make_async_copy(rows_v.at[lax.rem(NG - 1, 2)],
                          aggr_s.at[dst_v.at[NG - 1]],
                          sem_sc.at[lax.rem(NG - 1, 2)]).wait()
    plsc.subcore_barrier()
    pltpu.sync_copy(aggr_s.at[pl.ds(s * ZROWS, ZROWS)],
                    out_hbm.at[c, pl.ds(s * ZROWS, ZROWS)])


def _edge_call(h0, h1, src_p, dst_p, e0, e1, zeros):
    return pl.kernel(
        _edge_body,
        out_type=jax.ShapeDtypeStruct((NC, NPAD, HH), jnp.float32),
        mesh=_mesh,
        compiler_params=pltpu.CompilerParams(use_tc_tiling_on_sc=False),
        scratch_types=[
            pltpu.VMEM((EPS,), jnp.int32),
            pltpu.VMEM((NG, GRP), jnp.int32),
            pltpu.VMEM((2, GRP // 2, H), jnp.float32),
            pltpu.VMEM((2, GRP, HH), jnp.float32),
            pltpu.SemaphoreType.DMA((2,)),
            pltpu.SemaphoreType.DMA((2,)),
            pltpu.SemaphoreType.DMA((2,)),
            pltpu.VMEM_SHARED((NPAD, HH), jnp.float32),
        ],
    )(h0, h1, src_p, dst_p, e0, e1, zeros)


def _node_body(h_ref, a_ref, w1_ref, b1_ref, w2_ref, b2_ref, out_ref):
    t = h_ref[...] + a_ref[...]
    u = jnp.dot(t, w1_ref[...], preferred_element_type=jnp.float32) + b1_ref[...]
    u = u * jax.nn.sigmoid(u)
    v = jnp.dot(u, w2_ref[...], preferred_element_type=jnp.float32) + b2_ref[...]
    out_ref[...] = v * jax.nn.sigmoid(v)


_node_call = pl.pallas_call(
    _node_body,
    out_shape=jax.ShapeDtypeStruct((N, H), jnp.float32),
)


def kernel(z, edge_index, edge_attr, batch, emb, We, be, W1, b1, W2, b2):
    z = z.astype(jnp.int32)
    src = edge_index[0].astype(jnp.int32)
    dst = edge_index[1].astype(jnp.int32)
    ea = edge_attr.astype(jnp.float32)

    z_pad = jnp.concatenate([z, jnp.zeros((NPAD - N,), jnp.int32)])
    src_p = jnp.concatenate([src, jnp.zeros((EPAD - src.shape[0],), jnp.int32)])
    dst_p = jnp.concatenate([dst, jnp.full((EPAD - dst.shape[0],), N, jnp.int32)])
    dst_p = dst_p.reshape(NS, NG, GRP)
    ea_p = jnp.concatenate([ea, jnp.zeros((EPAD - ea.shape[0], NE), jnp.float32)])
    zeros = jnp.zeros((ZROWS, HH), jnp.float32)

    ea_pairs = ea_p.reshape(EPAD // 2, 2 * NE)
    wh = We.reshape(L, NE, NC, HH).transpose(0, 2, 1, 3)
    wz = jnp.zeros((L, NC, NE, HH), jnp.float32)
    wtop = jnp.concatenate([wh, wz], axis=3)
    wbot = jnp.concatenate([wz, wh], axis=3)
    w2cat = jnp.concatenate([wtop, wbot], axis=2)
    w2cat = w2cat.transpose(2, 0, 1, 3).reshape(2 * NE, L * NC * H)
    bh = be.reshape(L, NC, 1, HH)
    b2cat = jnp.concatenate([bh, bh], axis=3).reshape(1, L * NC * H)
    e_lc = _eproj(ea_pairs, w2cat, b2cat)

    x_pad = _emb_gather(emb, z_pad)
    h = x_pad[:N]
    for l in range(L):
        h0 = h[:, :HH]
        h1 = h[:, HH:]
        aggr2 = _edge_call(h0, h1, src_p, dst_p, e_lc[l * NC], e_lc[l * NC + 1],
                           zeros)
        a = jnp.concatenate([aggr2[0, :N], aggr2[1, :N]], axis=1)
        h = _node_call(h, a, W1[l], b1[l].reshape(1, H), W2[l], b2[l].reshape(1, H))
    return (h, batch)


# speedup vs baseline: 1.0091x; 1.0091x over previous
import jax
import jax.numpy as jnp
from jax import lax
from jax.experimental import pallas as pl
from jax.experimental.pallas import tpu as pltpu
from jax.experimental.pallas import tpu_sc as plsc

N = 10000
H = 128
HH = 64
NE = 4
L = 3

NLANES = 16
NC = 2
NS = 16
NW = NC * NS

NPAD = 10240
ROWS_PW = NPAD // NW
EGRP = 80
NEG = ROWS_PW // EGRP

GRP = 128
NG = 158
EPS = NG * GRP
EPAD = NS * EPS
ZROWS = NPAD // NS

EBLK = 4096
NEBLK = EPAD // EBLK

_mesh = plsc.VectorSubcoreMesh(core_axis_name="c", subcore_axis_name="s")


def _emb_body(emb_hbm, z_hbm, out_hbm, z_v, rows_v, sem):
    c = lax.axis_index("c")
    s = lax.axis_index("s")
    wid = s * NC + c
    base = wid * ROWS_PW
    pltpu.sync_copy(z_hbm.at[pl.ds(base, ROWS_PW)], z_v)

    def body(g, carry):
        pltpu.async_copy(emb_hbm.at[z_v.at[pl.ds(g * EGRP, EGRP)]], rows_v, sem).wait()
        pltpu.sync_copy(rows_v, out_hbm.at[pl.ds(base + g * EGRP, EGRP)])
        return carry

    lax.fori_loop(0, NEG, body, 0)


def _emb_gather(emb, z_pad):
    return pl.kernel(
        _emb_body,
        out_type=jax.ShapeDtypeStruct((NPAD, H), jnp.float32),
        mesh=_mesh,
        scratch_types=[
            pltpu.VMEM((ROWS_PW,), jnp.int32),
            pltpu.VMEM((EGRP, H), jnp.float32),
            pltpu.SemaphoreType.DMA,
        ],
    )(emb, z_pad)


def _eproj_body(ea_ref, w_ref, b_ref, *out_refs):
    e = jnp.dot(ea_ref[...], w_ref[...],
                preferred_element_type=jnp.float32) + b_ref[...]
    for i in range(L * NC):
        out_refs[i][...] = e[:, i * H:(i + 1) * H]


def _eproj(ea_pairs, w2cat, b2cat):
    return pl.pallas_call(
        _eproj_body,
        grid=(NEBLK,),
        in_specs=[
            pl.BlockSpec((EBLK // 2, 2 * NE), lambda i: (i, 0)),
            pl.BlockSpec((2 * NE, L * NC * H), lambda i: (0, 0)),
            pl.BlockSpec((1, L * NC * H), lambda i: (0, 0)),
        ],
        out_specs=[pl.BlockSpec((EBLK // 2, H), lambda i: (i, 0))] * (L * NC),
        out_shape=[jax.ShapeDtypeStruct((EPAD // 2, H), jnp.float32)] * (L * NC),
    )(ea_pairs, w2cat, b2cat)


def _edge_body(h0_hbm, h1_hbm, src_hbm, dst_hbm, e0_hbm, e1_hbm, zero_hbm,
               out_hbm, src_v, dst_v, e_v, rows_v, sem_r, sem_e, sem_sc, aggr_s):
    c = lax.axis_index("c")
    s = lax.axis_index("s")
    ebase = s * EPS

    pltpu.sync_copy(src_hbm.at[pl.ds(ebase, EPS)], src_v)
    pltpu.sync_copy(dst_hbm.at[s], dst_v)

    def issue_rows(g, slot):
        idx = src_v.at[pl.ds(g * GRP, GRP)]

        @pl.when(c == 0)
        def _():
            pltpu.async_copy(h0_hbm.at[idx], rows_v.at[slot], sem_r.at[slot])

        @pl.when(c == 1)
        def _():
            pltpu.async_copy(h1_hbm.at[idx], rows_v.at[slot], sem_r.at[slot])

    def issue_e(g, slot):
        sl = pl.ds((ebase + g * GRP) * HH, GRP * HH)

        @pl.when(c == 0)
        def _():
            pltpu.async_copy(e0_hbm.at[sl], e_v.at[slot], sem_e.at[slot])

        @pl.when(c == 1)
        def _():
            pltpu.async_copy(e1_hbm.at[sl], e_v.at[slot], sem_e.at[slot])

    issue_rows(0, 0)
    issue_e(0, 0)

    pltpu.sync_copy(zero_hbm, aggr_s.at[pl.ds(s * ZROWS, ZROWS)])
    plsc.subcore_barrier()

    def group(g, carry):
        slot = lax.rem(g, 2)
        nslot = 1 - slot
        pltpu.make_async_copy(h0_hbm.at[src_v.at[pl.ds(g * GRP, GRP)]],
                              rows_v.at[slot], sem_r.at[slot]).wait()
        pltpu.make_async_copy(e0_hbm.at[pl.ds((ebase + g * GRP) * HH, GRP * HH)],
                              e_v.at[slot], sem_e.at[slot]).wait()

        @pl.when(g >= 1)
        def _():
            pltpu.make_async_copy(rows_v.at[nslot], aggr_s.at[dst_v.at[g]],
                                  sem_sc.at[nslot]).wait()

        @pl.when(g + 1 < NG)
        def _():
            issue_rows(g + 1, nslot)
            issue_e(g + 1, nslot)

        def block(b, bcarry):
            for q in range(4):
                r = []
                for u in range(4):
                    i = b * NLANES + q * 4 + u
                    for j in range(HH // NLANES):
                        rv = rows_v[slot, i, pl.ds(j * NLANES, NLANES)]
                        ev = e_v[slot, pl.ds(i * HH + j * NLANES, NLANES)]
                        r.append(jnp.maximum(rv + ev, 0.0))
                idx = 0
                for u in range(4):
                    i = b * NLANES + q * 4 + u
                    for j in range(HH // NLANES):
                        rows_v[slot, i, pl.ds(j * NLANES, NLANES)] = r[idx]
                        idx += 1
            return bcarry

        lax.fori_loop(0, GRP // NLANES, block, 0)

        pltpu.async_copy(rows_v.at[slot], aggr_s.at[dst_v.at[g]],
                         sem_sc.at[slot], add=True)
        return carry

    lax.fori_loop(0, NG, group, 0)

    pltpu.make_async_copy(rows_v.at[lax.rem(NG - 1, 2)],
                          aggr_s.at[dst_v.at[NG - 1]],
                          sem_sc.at[lax.rem(NG - 1, 2)]).wait()
    plsc.subcore_barrier()
    pltpu.sync_copy(aggr_s.at[pl.ds(s * ZROWS, ZROWS)],
                    out_hbm.at[c, pl.ds(s * ZROWS, ZROWS)])


def _edge_call(h0, h1, src_p, dst_p, e0, e1, zeros):
    return pl.kernel(
        _edge_body,
        out_type=jax.ShapeDtypeStruct((NC, NPAD, HH), jnp.float32),
        mesh=_mesh,
        compiler_params=pltpu.CompilerParams(use_tc_tiling_on_sc=False),
        scratch_types=[
            pltpu.VMEM((EPS,), jnp.int32),
            pltpu.VMEM((NG, GRP), jnp.int32),
            pltpu.VMEM((2, GRP * HH), jnp.float32),
            pltpu.VMEM((2, GRP, HH), jnp.float32),
            pltpu.SemaphoreType.DMA((2,)),
            pltpu.SemaphoreType.DMA((2,)),
            pltpu.SemaphoreType.DMA((2,)),
            pltpu.VMEM_SHARED((NPAD, HH), jnp.float32),
        ],
    )(h0, h1, src_p, dst_p, e0, e1, zeros)


def _node_body(h_ref, a_ref, w1_ref, b1_ref, w2_ref, b2_ref, out_ref):
    t = h_ref[...] + a_ref[...]
    u = jnp.dot(t, w1_ref[...], preferred_element_type=jnp.float32) + b1_ref[...]
    u = u * jax.nn.sigmoid(u)
    v = jnp.dot(u, w2_ref[...], preferred_element_type=jnp.float32) + b2_ref[...]
    out_ref[...] = v * jax.nn.sigmoid(v)


_node_call = pl.pallas_call(
    _node_body,
    out_shape=jax.ShapeDtypeStruct((N, H), jnp.float32),
)


def kernel(z, edge_index, edge_attr, batch, emb, We, be, W1, b1, W2, b2):
    z = z.astype(jnp.int32)
    src = edge_index[0].astype(jnp.int32)
    dst = edge_index[1].astype(jnp.int32)
    ea = edge_attr.astype(jnp.float32)

    z_pad = jnp.concatenate([z, jnp.zeros((NPAD - N,), jnp.int32)])
    src_p = jnp.concatenate([src, jnp.zeros((EPAD - src.shape[0],), jnp.int32)])
    dst_p = jnp.concatenate([dst, jnp.full((EPAD - dst.shape[0],), N, jnp.int32)])
    dst_p = dst_p.reshape(NS, NG, GRP)
    ea_p = jnp.concatenate([ea, jnp.zeros((EPAD - ea.shape[0], NE), jnp.float32)])
    zeros = jnp.zeros((ZROWS, HH), jnp.float32)

    ea_pairs = ea_p.reshape(EPAD // 2, 2 * NE)
    wh = We.reshape(L, NE, NC, HH).transpose(0, 2, 1, 3)
    wz = jnp.zeros((L, NC, NE, HH), jnp.float32)
    wtop = jnp.concatenate([wh, wz], axis=3)
    wbot = jnp.concatenate([wz, wh], axis=3)
    w2cat = jnp.concatenate([wtop, wbot], axis=2)
    w2cat = w2cat.transpose(2, 0, 1, 3).reshape(2 * NE, L * NC * H)
    bh = be.reshape(L, NC, 1, HH)
    b2cat = jnp.concatenate([bh, bh], axis=3).reshape(1, L * NC * H)
    e_lc = _eproj(ea_pairs, w2cat, b2cat)

    x_pad = _emb_gather(emb, z_pad)
    h = x_pad[:N]
    for l in range(L):
        h0 = h[:, :HH]
        h1 = h[:, HH:]
        aggr2 = _edge_call(h0, h1, src_p, dst_p,
                           e_lc[l * NC].reshape(-1),
                           e_lc[l * NC + 1].reshape(-1), zeros)
        a = jnp.concatenate([aggr2[0, :N], aggr2[1, :N]], axis=1)
        h = _node_call(h, a, W1[l], b1[l].reshape(1, H), W2[l], b2[l].reshape(1, H))
    return (h, batch)


# speedup vs baseline: 1.0429x; 1.0335x over previous
import jax
import jax.numpy as jnp
import numpy as np
from jax import lax
from jax.experimental import pallas as pl
from jax.experimental.pallas import tpu as pltpu
from jax.experimental.pallas import tpu_sc as plsc

N = 10000
H = 128
HH = 64
HW = HH // 2
NE = 4
L = 3

NLANES = 16
NC = 2
NS = 16
NW = NC * NS

NPAD = 10240
ROWS_PW = NPAD // NW
EGRP = 80
NEG = ROWS_PW // EGRP

GRP = 128
NG = 158
EPS = NG * GRP
EPAD = NS * EPS
ZROWS = NPAD // NS

EBLK = 4096
NEBLK = EPAD // EBLK
EW = L * NC * HH

_lc = np.arange(L * NC)[:, None, None, None]
_p = np.arange(2)[None, :, None, None]
_k = np.arange(2)[None, None, :, None]
_j = np.arange(16)[None, None, None, :]
_BASE = (_lc * H + _p * HH + _k * 32 + _j).reshape(-1)
COLPERM = np.concatenate([_BASE, _BASE + 16])

_c2 = np.arange(NC)[:, None, None]
_k2 = np.arange(2)[None, :, None]
_j2 = np.arange(16)[None, None, :]
HSEL = (_c2 * HH + _k2 * 32 + _j2).reshape(-1)

_mesh = plsc.VectorSubcoreMesh(core_axis_name="c", subcore_axis_name="s")


def _pack_words(lo, hi):
    lo_u = jax.lax.bitcast_convert_type(lo.astype(jnp.bfloat16).astype(jnp.float32),
                                        jnp.uint32)
    hi_u = jax.lax.bitcast_convert_type(hi.astype(jnp.bfloat16).astype(jnp.float32),
                                        jnp.uint32)
    return jax.lax.bitcast_convert_type((lo_u >> 16) | hi_u, jnp.int32)


def _emb_body(emb_hbm, z_hbm, out_hbm, z_v, rows_v, sem):
    c = lax.axis_index("c")
    s = lax.axis_index("s")
    wid = s * NC + c
    base = wid * ROWS_PW
    pltpu.sync_copy(z_hbm.at[pl.ds(base, ROWS_PW)], z_v)

    def body(g, carry):
        pltpu.async_copy(emb_hbm.at[z_v.at[pl.ds(g * EGRP, EGRP)]], rows_v, sem).wait()
        pltpu.sync_copy(rows_v, out_hbm.at[pl.ds(base + g * EGRP, EGRP)])
        return carry

    lax.fori_loop(0, NEG, body, 0)


def _emb_gather(emb, z_pad):
    return pl.kernel(
        _emb_body,
        out_type=jax.ShapeDtypeStruct((NPAD, H), jnp.float32),
        mesh=_mesh,
        scratch_types=[
            pltpu.VMEM((ROWS_PW,), jnp.int32),
            pltpu.VMEM((EGRP, H), jnp.float32),
            pltpu.SemaphoreType.DMA,
        ],
    )(emb, z_pad)


def _eproj_body(ea_ref, w_ref, b_ref, out_ref):
    e = jnp.dot(ea_ref[...], w_ref[...],
                preferred_element_type=jnp.float32) + b_ref[...]
    out_ref[...] = _pack_words(e[:, :EW], e[:, EW:])


def _eproj(ea_pairs, w2cat, b2cat):
    return pl.pallas_call(
        _eproj_body,
        grid=(NEBLK,),
        in_specs=[
            pl.BlockSpec((EBLK // 2, 2 * NE), lambda i: (i, 0)),
            pl.BlockSpec((2 * NE, L * NC * H), lambda i: (0, 0)),
            pl.BlockSpec((1, L * NC * H), lambda i: (0, 0)),
        ],
        out_specs=pl.BlockSpec((EBLK // 2, EW), lambda i: (i, 0)),
        out_shape=jax.ShapeDtypeStruct((EPAD // 2, EW), jnp.int32),
    )(ea_pairs, w2cat, b2cat)


def _make_edge_body(l):
    lcbase = l * NC

    def _edge_body(h0_hbm, h1_hbm, src_hbm, dst_hbm, e_hbm, zero_hbm,
                   out_hbm, src_v, dst_v, e_v, rows_v, m_v, sem_r, sem_e,
                   sem_sc, aggr_s):
        c = lax.axis_index("c")
        s = lax.axis_index("s")
        ebase = s * EPS
        ecol = (lcbase + c) * (2 * HW)

        pltpu.sync_copy(src_hbm.at[pl.ds(ebase, EPS)], src_v)
        pltpu.sync_copy(dst_hbm.at[s], dst_v)

        def issue_rows(g, slot):
            idx = src_v.at[pl.ds(g * GRP, GRP)]

            @pl.when(c == 0)
            def _():
                pltpu.async_copy(h0_hbm.at[idx], rows_v.at[slot], sem_r.at[slot])

            @pl.when(c == 1)
            def _():
                pltpu.async_copy(h1_hbm.at[idx], rows_v.at[slot], sem_r.at[slot])

        def e_src(g):
            return e_hbm.at[pl.ds((ebase + g * GRP) // 2, GRP // 2),
                            pl.ds(ecol, 2 * HW)]

        def issue_e(g, slot):
            pltpu.async_copy(e_src(g), e_v.at[slot], sem_e.at[slot])

        issue_rows(0, 0)
        issue_e(0, 0)

        pltpu.sync_copy(zero_hbm, aggr_s.at[pl.ds(s * ZROWS, ZROWS)])
        plsc.subcore_barrier()

        def group(g, carry):
            slot = lax.rem(g, 2)
            nslot = 1 - slot
            pltpu.make_async_copy(h0_hbm.at[src_v.at[pl.ds(g * GRP, GRP)]],
                                  rows_v.at[slot], sem_r.at[slot]).wait()
            pltpu.make_async_copy(e_src(g), e_v.at[slot], sem_e.at[slot]).wait()

            @pl.when(g >= 1)
            def _():
                pltpu.make_async_copy(m_v.at[nslot], aggr_s.at[dst_v.at[g]],
                                      sem_sc.at[nslot]).wait()

            @pl.when(g + 1 < NG)
            def _():
                issue_rows(g + 1, nslot)
                issue_e(g + 1, nslot)

            def block(b, bcarry):
                for q in range(4):
                    ms = []
                    for u in range(4):
                        i = b * NLANES + q * 4 + u
                        prow = b * 8 + (q * 4 + u) // 2
                        pcol = (u % 2) * HW
                        for k in range(2):
                            rw = rows_v[slot, i, pl.ds(k * NLANES, NLANES)]
                            ew = e_v[slot, prow,
                                     pl.ds(pcol + k * NLANES, NLANES)]
                            rlo = plsc.bitcast(rw << 16, jnp.float32)
                            rhi = plsc.bitcast(rw & -65536, jnp.float32)
                            elo = plsc.bitcast(ew << 16, jnp.float32)
                            ehi = plsc.bitcast(ew & -65536, jnp.float32)
                            ms.append(jnp.maximum(rlo + elo, 0.0))
                            ms.append(jnp.maximum(rhi + ehi, 0.0))
                    idx = 0
                    for u in range(4):
                        i = b * NLANES + q * 4 + u
                        for k in range(2):
                            m_v[slot, i, pl.ds(k * 32, NLANES)] = ms[idx]
                            m_v[slot, i, pl.ds(k * 32 + NLANES, NLANES)] = \
                                ms[idx + 1]
                            idx += 2
                return bcarry

            lax.fori_loop(0, GRP // NLANES, block, 0)

            pltpu.async_copy(m_v.at[slot], aggr_s.at[dst_v.at[g]],
                             sem_sc.at[slot], add=True)
            return carry

        lax.fori_loop(0, NG, group, 0)

        pltpu.make_async_copy(m_v.at[lax.rem(NG - 1, 2)],
                              aggr_s.at[dst_v.at[NG - 1]],
                              sem_sc.at[lax.rem(NG - 1, 2)]).wait()
        plsc.subcore_barrier()
        pltpu.sync_copy(aggr_s.at[pl.ds(s * ZROWS, ZROWS)],
                        out_hbm.at[c, pl.ds(s * ZROWS, ZROWS)])

    return _edge_body


def _edge_call(l, hb0, hb1, src_p, dst_p, e_pk, zeros):
    return pl.kernel(
        _make_edge_body(l),
        out_type=jax.ShapeDtypeStruct((NC, NPAD, HH), jnp.float32),
        mesh=_mesh,
        compiler_params=pltpu.CompilerParams(use_tc_tiling_on_sc=False,
                                             needs_layout_passes=False),
        scratch_types=[
            pltpu.VMEM((EPS,), jnp.int32),
            pltpu.VMEM((NG, GRP), jnp.int32),
            pltpu.VMEM((2, GRP // 2, 2 * HW), jnp.int32),
            pltpu.VMEM((2, GRP, HW), jnp.int32),
            pltpu.VMEM((2, GRP, HH), jnp.float32),
            pltpu.SemaphoreType.DMA((2,)),
            pltpu.SemaphoreType.DMA((2,)),
            pltpu.SemaphoreType.DMA((2,)),
            pltpu.VMEM_SHARED((NPAD, HH), jnp.float32),
        ],
    )(hb0, hb1, src_p, dst_p, e_pk, zeros)


def _node_body(h_ref, aggr_ref, w1_ref, b1_ref, w2_ref, b2_ref, out_ref):
    a = jnp.concatenate([aggr_ref[0, :N, :], aggr_ref[1, :N, :]], axis=1)
    t = h_ref[...] + a
    u = jnp.dot(t, w1_ref[...], preferred_element_type=jnp.float32) + b1_ref[...]
    u = u * jax.nn.sigmoid(u)
    v = jnp.dot(u, w2_ref[...], preferred_element_type=jnp.float32) + b2_ref[...]
    out_ref[...] = v * jax.nn.sigmoid(v)


_node_call = pl.pallas_call(
    _node_body,
    out_shape=jax.ShapeDtypeStruct((N, H), jnp.float32),
)


def _pack_h(h):
    hw = _pack_words(jnp.take(h, jnp.asarray(HSEL), axis=1),
                     jnp.take(h, jnp.asarray(HSEL + 16), axis=1))
    return hw[:, :HW], hw[:, HW:]


def kernel(z, edge_index, edge_attr, batch, emb, We, be, W1, b1, W2, b2):
    z = z.astype(jnp.int32)
    src = edge_index[0].astype(jnp.int32)
    dst = edge_index[1].astype(jnp.int32)
    ea = edge_attr.astype(jnp.float32)

    z_pad = jnp.concatenate([z, jnp.zeros((NPAD - N,), jnp.int32)])
    src_p = jnp.concatenate([src, jnp.zeros((EPAD - src.shape[0],), jnp.int32)])
    dst_p = jnp.concatenate([dst, jnp.full((EPAD - dst.shape[0],), N, jnp.int32)])
    dst_p = dst_p.reshape(NS, NG, GRP)
    ea_p = jnp.concatenate([ea, jnp.zeros((EPAD - ea.shape[0], NE), jnp.float32)])
    zeros = jnp.zeros((ZROWS, HH), jnp.float32)

    ea_pairs = ea_p.reshape(EPAD // 2, 2 * NE)
    wh = We.reshape(L, NE, NC, HH).transpose(0, 2, 1, 3)
    wz = jnp.zeros((L, NC, NE, HH), jnp.float32)
    wtop = jnp.concatenate([wh, wz], axis=3)
    wbot = jnp.concatenate([wz, wh], axis=3)
    w2cat = jnp.concatenate([wtop, wbot], axis=2)
    w2cat = w2cat.transpose(2, 0, 1, 3).reshape(2 * NE, L * NC * H)[:, COLPERM]
    bh = be.reshape(L, NC, 1, HH)
    b2cat = jnp.concatenate([bh, bh], axis=3).reshape(1, L * NC * H)[:, COLPERM]
    e_pk = _eproj(ea_pairs, w2cat, b2cat)

    x_pad = _emb_gather(emb, z_pad)
    h = x_pad[:N]
    for l in range(L):
        hb0, hb1 = _pack_h(h)
        aggr2 = _edge_call(l, hb0, hb1, src_p, dst_p, e_pk, zeros)
        h = _node_call(h, aggr2, W1[l], b1[l].reshape(1, H),
                       W2[l], b2[l].reshape(1, H))
    return (h, batch)


# speedup vs baseline: 1.1530x; 1.1056x over previous
import jax
import jax.numpy as jnp
import numpy as np
from jax import lax
from jax.experimental import pallas as pl
from jax.experimental.pallas import tpu as pltpu
from jax.experimental.pallas import tpu_sc as plsc

N = 10000
H = 128
HH = 64
HW = HH // 2
NE = 4
L = 3

NLANES = 16
NC = 2
NS = 16
NW = NC * NS

NPAD = 10240
ROWS_PW = NPAD // NW
EGRP = 80
NEG = ROWS_PW // EGRP

GRP = 128
NG = 158
EPS = NG * GRP
EPAD = NS * EPS
ZROWS = NPAD // NS

EBLK = 4096
NEBLK = EPAD // EBLK
EW = L * NC * HH

_lc = np.arange(L * NC)[:, None, None, None]
_p = np.arange(2)[None, :, None, None]
_k = np.arange(2)[None, None, :, None]
_j = np.arange(16)[None, None, None, :]
_BASE = (_lc * H + _p * HH + _k * 32 + _j).reshape(-1)
COLPERM = np.concatenate([_BASE, _BASE + 16])

_c2 = np.arange(NC)[:, None, None]
_k2 = np.arange(2)[None, :, None]
_j2 = np.arange(16)[None, None, :]
HSEL = (_c2 * HH + _k2 * 32 + _j2).reshape(-1)

_mesh = plsc.VectorSubcoreMesh(core_axis_name="c", subcore_axis_name="s")


def _pack_words(lo, hi):
    lo_u = jax.lax.bitcast_convert_type(lo.astype(jnp.bfloat16).astype(jnp.float32),
                                        jnp.uint32)
    hi_u = jax.lax.bitcast_convert_type(hi.astype(jnp.bfloat16).astype(jnp.float32),
                                        jnp.uint32)
    return jax.lax.bitcast_convert_type((lo_u >> 16) | hi_u, jnp.int32)


def _emb_body(emb_hbm, z_hbm, out_hbm, z_v, rows_v, sem):
    c = lax.axis_index("c")
    s = lax.axis_index("s")
    wid = s * NC + c
    base = wid * ROWS_PW
    pltpu.sync_copy(z_hbm.at[pl.ds(base, ROWS_PW)], z_v)

    def body(g, carry):
        pltpu.async_copy(emb_hbm.at[z_v.at[pl.ds(g * EGRP, EGRP)]], rows_v, sem).wait()
        pltpu.sync_copy(rows_v, out_hbm.at[pl.ds(base + g * EGRP, EGRP)])
        return carry

    lax.fori_loop(0, NEG, body, 0)


def _emb_gather(emb, z_pad):
    return pl.kernel(
        _emb_body,
        out_type=jax.ShapeDtypeStruct((NPAD, H), jnp.float32),
        mesh=_mesh,
        scratch_types=[
            pltpu.VMEM((ROWS_PW,), jnp.int32),
            pltpu.VMEM((EGRP, H), jnp.float32),
            pltpu.SemaphoreType.DMA,
        ],
    )(emb, z_pad)


def _eproj_body(ea_ref, w_ref, b_ref, out_ref):
    e = jnp.dot(ea_ref[...], w_ref[...],
                preferred_element_type=jnp.float32) + b_ref[...]
    out_ref[...] = _pack_words(e[:, :EW], e[:, EW:])


def _eproj(ea_pairs, w2cat, b2cat):
    return pl.pallas_call(
        _eproj_body,
        grid=(NEBLK,),
        in_specs=[
            pl.BlockSpec((EBLK // 2, 2 * NE), lambda i: (i, 0)),
            pl.BlockSpec((2 * NE, L * NC * H), lambda i: (0, 0)),
            pl.BlockSpec((1, L * NC * H), lambda i: (0, 0)),
        ],
        out_specs=pl.BlockSpec((EBLK // 2, EW), lambda i: (i, 0)),
        out_shape=jax.ShapeDtypeStruct((EPAD // 2, EW), jnp.int32),
    )(ea_pairs, w2cat, b2cat)


def _make_edge_body(l):
    lcbase = l * NC

    def _edge_body(h0_hbm, h1_hbm, src_hbm, dst_hbm, e_hbm, zero_hbm,
                   out_hbm, src_v, dst_v, e_v, rows_v, m_v, sem_r, sem_e,
                   sem_sc, aggr_s):
        c = lax.axis_index("c")
        s = lax.axis_index("s")
        ebase = s * EPS
        ecol = (lcbase + c) * (2 * HW)

        pltpu.sync_copy(src_hbm.at[pl.ds(ebase, EPS)], src_v)
        pltpu.sync_copy(dst_hbm.at[s], dst_v)

        def issue_rows(g, slot):
            idx = src_v.at[pl.ds(g * GRP, GRP)]

            @pl.when(c == 0)
            def _():
                pltpu.async_copy(h0_hbm.at[idx], rows_v.at[slot], sem_r.at[slot])

            @pl.when(c == 1)
            def _():
                pltpu.async_copy(h1_hbm.at[idx], rows_v.at[slot], sem_r.at[slot])

        def e_src(g):
            return e_hbm.at[pl.ds((ebase + g * GRP) // 2, GRP // 2),
                            pl.ds(ecol, 2 * HW)]

        def issue_e(g, slot):
            pltpu.async_copy(e_src(g), e_v.at[slot], sem_e.at[slot])

        issue_rows(0, 0)
        issue_e(0, 0)

        pltpu.sync_copy(zero_hbm, aggr_s.at[pl.ds(s * ZROWS, ZROWS)])
        plsc.subcore_barrier()

        def group(g, carry):
            slot = lax.rem(g, 2)
            nslot = 1 - slot
            pltpu.make_async_copy(h0_hbm.at[src_v.at[pl.ds(g * GRP, GRP)]],
                                  rows_v.at[slot], sem_r.at[slot]).wait()
            pltpu.make_async_copy(e_src(g), e_v.at[slot], sem_e.at[slot]).wait()

            @pl.when(g >= 1)
            def _():
                pltpu.make_async_copy(m_v.at[nslot], aggr_s.at[dst_v.at[g]],
                                      sem_sc.at[nslot]).wait()

            @pl.when(g + 1 < NG)
            def _():
                issue_rows(g + 1, nslot)
                issue_e(g + 1, nslot)

            def block(b, bcarry):
                for q in range(4):
                    ms = []
                    for u in range(4):
                        i = b * NLANES + q * 4 + u
                        prow = b * 8 + (q * 4 + u) // 2
                        pcol = (u % 2) * HW
                        for k in range(2):
                            rw = rows_v[slot, i, pl.ds(k * NLANES, NLANES)]
                            ew = e_v[slot, prow,
                                     pl.ds(pcol + k * NLANES, NLANES)]
                            rlo = plsc.bitcast(rw << 16, jnp.float32)
                            rhi = plsc.bitcast(rw & -65536, jnp.float32)
                            elo = plsc.bitcast(ew << 16, jnp.float32)
                            ehi = plsc.bitcast(ew & -65536, jnp.float32)
                            ms.append(jnp.maximum(rlo + elo, 0.0))
                            ms.append(jnp.maximum(rhi + ehi, 0.0))
                    idx = 0
                    for u in range(4):
                        i = b * NLANES + q * 4 + u
                        for k in range(2):
                            m_v[slot, i, pl.ds(k * 32, NLANES)] = ms[idx]
                            m_v[slot, i, pl.ds(k * 32 + NLANES, NLANES)] = \
                                ms[idx + 1]
                            idx += 2
                return bcarry

            lax.fori_loop(0, GRP // NLANES, block, 0)

            pltpu.async_copy(m_v.at[slot], aggr_s.at[dst_v.at[g]],
                             sem_sc.at[slot], add=True)
            return carry

        lax.fori_loop(0, NG, group, 0)

        pltpu.make_async_copy(m_v.at[lax.rem(NG - 1, 2)],
                              aggr_s.at[dst_v.at[NG - 1]],
                              sem_sc.at[lax.rem(NG - 1, 2)]).wait()
        plsc.subcore_barrier()
        pltpu.sync_copy(aggr_s.at[pl.ds(s * ZROWS, ZROWS)],
                        out_hbm.at[c, pl.ds(s * ZROWS, ZROWS)])

    return _edge_body


def _edge_call(l, hb0, hb1, src_p, dst_p, e_pk, zeros):
    return pl.kernel(
        _make_edge_body(l),
        out_type=jax.ShapeDtypeStruct((NC, NPAD, HH), jnp.float32),
        mesh=_mesh,
        compiler_params=pltpu.CompilerParams(use_tc_tiling_on_sc=False,
                                             needs_layout_passes=False),
        scratch_types=[
            pltpu.VMEM((EPS,), jnp.int32),
            pltpu.VMEM((NG, GRP), jnp.int32),
            pltpu.VMEM((2, GRP // 2, 2 * HW), jnp.int32),
            pltpu.VMEM((2, GRP, HW), jnp.int32),
            pltpu.VMEM((2, GRP, HH), jnp.float32),
            pltpu.SemaphoreType.DMA((2,)),
            pltpu.SemaphoreType.DMA((2,)),
            pltpu.SemaphoreType.DMA((2,)),
            pltpu.VMEM_SHARED((NPAD, HH), jnp.float32),
        ],
    )(hb0, hb1, src_p, dst_p, e_pk, zeros)


def _node_body(h_ref, aggr_ref, w1_ref, b1_ref, w2_ref, b2_ref, out_ref):
    a = jnp.concatenate([aggr_ref[0, :N, :], aggr_ref[1, :N, :]], axis=1)
    t = h_ref[...] + a
    u = jnp.dot(t, w1_ref[...], preferred_element_type=jnp.float32) + b1_ref[...]
    u = u * jax.nn.sigmoid(u)
    v = jnp.dot(u, w2_ref[...], preferred_element_type=jnp.float32) + b2_ref[...]
    out_ref[...] = v * jax.nn.sigmoid(v)


_node_call = pl.pallas_call(
    _node_body,
    out_shape=jax.ShapeDtypeStruct((N, H), jnp.float32),
)


def _pack_h(h):
    h4 = h.reshape(N, 4, 32)
    lo = h4[:, :, :NLANES].reshape(N, HH)
    hi = h4[:, :, NLANES:].reshape(N, HH)
    hw = _pack_words(lo, hi)
    return hw[:, :HW], hw[:, HW:]


def kernel(z, edge_index, edge_attr, batch, emb, We, be, W1, b1, W2, b2):
    z = z.astype(jnp.int32)
    src = edge_index[0].astype(jnp.int32)
    dst = edge_index[1].astype(jnp.int32)
    ea = edge_attr.astype(jnp.float32)

    z_pad = jnp.concatenate([z, jnp.zeros((NPAD - N,), jnp.int32)])
    pad_n = EPAD - src.shape[0]
    pad_i = jnp.arange(pad_n, dtype=jnp.int32)
    src_p = jnp.concatenate([src, pad_i % N])
    dst_p = jnp.concatenate([dst, N + pad_i % (NPAD - N)])
    dst_p = dst_p.reshape(NS, NG, GRP)
    ea_p = jnp.concatenate([ea, jnp.zeros((EPAD - ea.shape[0], NE), jnp.float32)])
    zeros = jnp.zeros((ZROWS, HH), jnp.float32)

    ea_pairs = ea_p.reshape(EPAD // 2, 2 * NE)
    wh = We.reshape(L, NE, NC, HH).transpose(0, 2, 1, 3)
    wz = jnp.zeros((L, NC, NE, HH), jnp.float32)
    wtop = jnp.concatenate([wh, wz], axis=3)
    wbot = jnp.concatenate([wz, wh], axis=3)
    w2cat = jnp.concatenate([wtop, wbot], axis=2)
    w2cat = w2cat.transpose(2, 0, 1, 3).reshape(2 * NE, L * NC * H)[:, COLPERM]
    bh = be.reshape(L, NC, 1, HH)
    b2cat = jnp.concatenate([bh, bh], axis=3).reshape(1, L * NC * H)[:, COLPERM]
    e_pk = _eproj(ea_pairs, w2cat, b2cat)

    x_pad = _emb_gather(emb, z_pad)
    h = x_pad[:N]
    for l in range(L):
        hb0, hb1 = _pack_h(h)
        aggr2 = _edge_call(l, hb0, hb1, src_p, dst_p, e_pk, zeros)
        h = _node_call(h, aggr2, W1[l], b1[l].reshape(1, H),
                       W2[l], b2[l].reshape(1, H))
    return (h, batch)


# speedup vs baseline: 1.1985x; 1.0394x over previous
import jax
import jax.numpy as jnp
import numpy as np
from jax import lax
from jax.experimental import pallas as pl
from jax.experimental.pallas import tpu as pltpu
from jax.experimental.pallas import tpu_sc as plsc

N = 10000
H = 128
HH = 64
HW = HH // 2
NE = 4
L = 3

NLANES = 16
NC = 2
NS = 16
NW = NC * NS

NPAD = 10240
ROWS_PW = NPAD // NW
EGRP = 80
NEG = ROWS_PW // EGRP

GRP = 128
NG = 158
EPS = NG * GRP
EPAD = NS * EPS
ZROWS = NPAD // NS

EBLK = 4096
NEBLK = EPAD // EBLK
EW = L * NC * HH

_lc = np.arange(L * NC)[:, None, None, None]
_p = np.arange(2)[None, :, None, None]
_k = np.arange(2)[None, None, :, None]
_j = np.arange(16)[None, None, None, :]
_BASE = (_lc * H + _p * HH + _k * 32 + _j).reshape(-1)
COLPERM = np.concatenate([_BASE, _BASE + 16])

_c2 = np.arange(NC)[:, None, None]
_k2 = np.arange(2)[None, :, None]
_j2 = np.arange(16)[None, None, :]
HSEL = (_c2 * HH + _k2 * 32 + _j2).reshape(-1)

_mesh = plsc.VectorSubcoreMesh(core_axis_name="c", subcore_axis_name="s")


def _pack_words(lo, hi):
    lo_u = jax.lax.bitcast_convert_type(lo.astype(jnp.bfloat16).astype(jnp.float32),
                                        jnp.uint32)
    hi_u = jax.lax.bitcast_convert_type(hi.astype(jnp.bfloat16).astype(jnp.float32),
                                        jnp.uint32)
    return jax.lax.bitcast_convert_type((lo_u >> 16) | hi_u, jnp.int32)


def _emb_body(emb_hbm, z_hbm, out_hbm, z_v, rows_v, sem):
    c = lax.axis_index("c")
    s = lax.axis_index("s")
    wid = s * NC + c
    base = wid * ROWS_PW
    pltpu.sync_copy(z_hbm.at[pl.ds(base, ROWS_PW)], z_v)

    def body(g, carry):
        pltpu.async_copy(emb_hbm.at[z_v.at[pl.ds(g * EGRP, EGRP)]], rows_v, sem).wait()
        pltpu.sync_copy(rows_v, out_hbm.at[pl.ds(base + g * EGRP, EGRP)])
        return carry

    lax.fori_loop(0, NEG, body, 0)


def _emb_gather(emb, z_pad):
    return pl.kernel(
        _emb_body,
        out_type=jax.ShapeDtypeStruct((NPAD, H), jnp.float32),
        mesh=_mesh,
        scratch_types=[
            pltpu.VMEM((ROWS_PW,), jnp.int32),
            pltpu.VMEM((EGRP, H), jnp.float32),
            pltpu.SemaphoreType.DMA,
        ],
    )(emb, z_pad)


def _eproj_body(ea_ref, w_ref, b_ref, out_ref):
    e = jnp.dot(ea_ref[...], w_ref[...],
                preferred_element_type=jnp.float32) + b_ref[...]
    out_ref[...] = _pack_words(e[:, :EW], e[:, EW:])


def _eproj(ea_pairs, w2cat, b2cat):
    return pl.pallas_call(
        _eproj_body,
        grid=(NEBLK,),
        in_specs=[
            pl.BlockSpec((EBLK // 2, 2 * NE), lambda i: (i, 0)),
            pl.BlockSpec((2 * NE, L * NC * H), lambda i: (0, 0)),
            pl.BlockSpec((1, L * NC * H), lambda i: (0, 0)),
        ],
        out_specs=pl.BlockSpec((EBLK // 2, EW), lambda i: (i, 0)),
        out_shape=jax.ShapeDtypeStruct((EPAD // 2, EW), jnp.int32),
    )(ea_pairs, w2cat, b2cat)


def _make_edge_body(l):
    lcbase = l * NC

    def _edge_body(h0_hbm, h1_hbm, src_hbm, dst_hbm, e_hbm, zero_hbm,
                   out_hbm, src_v, dst_v, e_v, rows_v, m_v, sem_r, sem_e,
                   sem_sc, aggr_s):
        c = lax.axis_index("c")
        s = lax.axis_index("s")
        ebase = s * EPS
        ecol = (lcbase + c) * (2 * HW)

        pltpu.sync_copy(src_hbm.at[pl.ds(ebase, EPS)], src_v)
        pltpu.sync_copy(dst_hbm.at[s], dst_v)

        def issue_rows(g, slot):
            idx = src_v.at[pl.ds(g * GRP, GRP)]

            @pl.when(c == 0)
            def _():
                pltpu.async_copy(h0_hbm.at[idx], rows_v.at[slot], sem_r.at[slot])

            @pl.when(c == 1)
            def _():
                pltpu.async_copy(h1_hbm.at[idx], rows_v.at[slot], sem_r.at[slot])

        def e_src(g):
            return e_hbm.at[pl.ds((ebase + g * GRP) // 2, GRP // 2),
                            pl.ds(ecol, 2 * HW)]

        def issue_e(g, slot):
            pltpu.async_copy(e_src(g), e_v.at[slot], sem_e.at[slot])

        issue_rows(0, 0)
        issue_e(0, 0)

        pltpu.sync_copy(zero_hbm, aggr_s.at[pl.ds(s * ZROWS, ZROWS)])
        plsc.subcore_barrier()

        def group(g, carry):
            slot = lax.rem(g, 2)
            nslot = 1 - slot
            pltpu.make_async_copy(h0_hbm.at[src_v.at[pl.ds(g * GRP, GRP)]],
                                  rows_v.at[slot], sem_r.at[slot]).wait()
            pltpu.make_async_copy(e_src(g), e_v.at[slot], sem_e.at[slot]).wait()

            @pl.when(g >= 1)
            def _():
                pltpu.make_async_copy(m_v.at[nslot], aggr_s.at[dst_v.at[g]],
                                      sem_sc.at[nslot]).wait()

            @pl.when(g + 1 < NG)
            def _():
                issue_rows(g + 1, nslot)
                issue_e(g + 1, nslot)

            @plsc.parallel_loop(0, GRP // NLANES)
            def block(b):
                for q in range(4):
                    ms = []
                    for u in range(4):
                        i = b * NLANES + q * 4 + u
                        prow = b * 8 + (q * 4 + u) // 2
                        pcol = (u % 2) * HW
                        for k in range(2):
                            rw = rows_v[slot, i, pl.ds(k * NLANES, NLANES)]
                            ew = e_v[slot, prow,
                                     pl.ds(pcol + k * NLANES, NLANES)]
                            rlo = plsc.bitcast(rw << 16, jnp.float32)
                            rhi = plsc.bitcast(rw & -65536, jnp.float32)
                            elo = plsc.bitcast(ew << 16, jnp.float32)
                            ehi = plsc.bitcast(ew & -65536, jnp.float32)
                            ms.append(jnp.maximum(rlo + elo, 0.0))
                            ms.append(jnp.maximum(rhi + ehi, 0.0))
                    idx = 0
                    for u in range(4):
                        i = b * NLANES + q * 4 + u
                        for k in range(2):
                            m_v[slot, i, pl.ds(k * 32, NLANES)] = ms[idx]
                            m_v[slot, i, pl.ds(k * 32 + NLANES, NLANES)] = \
                                ms[idx + 1]
                            idx += 2

            pltpu.async_copy(m_v.at[slot], aggr_s.at[dst_v.at[g]],
                             sem_sc.at[slot], add=True)
            return carry

        lax.fori_loop(0, NG, group, 0)

        pltpu.make_async_copy(m_v.at[lax.rem(NG - 1, 2)],
                              aggr_s.at[dst_v.at[NG - 1]],
                              sem_sc.at[lax.rem(NG - 1, 2)]).wait()
        plsc.subcore_barrier()
        pltpu.sync_copy(aggr_s.at[pl.ds(s * ZROWS, ZROWS)],
                        out_hbm.at[c, pl.ds(s * ZROWS, ZROWS)])

    return _edge_body


def _edge_call(l, hb0, hb1, src_p, dst_p, e_pk, zeros):
    return pl.kernel(
        _make_edge_body(l),
        out_type=jax.ShapeDtypeStruct((NC, NPAD, HH), jnp.float32),
        mesh=_mesh,
        compiler_params=pltpu.CompilerParams(use_tc_tiling_on_sc=False,
                                             needs_layout_passes=False),
        scratch_types=[
            pltpu.VMEM((EPS,), jnp.int32),
            pltpu.VMEM((NG, GRP), jnp.int32),
            pltpu.VMEM((2, GRP // 2, 2 * HW), jnp.int32),
            pltpu.VMEM((2, GRP, HW), jnp.int32),
            pltpu.VMEM((2, GRP, HH), jnp.float32),
            pltpu.SemaphoreType.DMA((2,)),
            pltpu.SemaphoreType.DMA((2,)),
            pltpu.SemaphoreType.DMA((2,)),
            pltpu.VMEM_SHARED((NPAD, HH), jnp.float32),
        ],
    )(hb0, hb1, src_p, dst_p, e_pk, zeros)


def _node_body(h_ref, aggr_ref, w1_ref, b1_ref, w2_ref, b2_ref, out_ref):
    a = jnp.concatenate([aggr_ref[0, :N, :], aggr_ref[1, :N, :]], axis=1)
    t = h_ref[...] + a
    u = jnp.dot(t, w1_ref[...], preferred_element_type=jnp.float32) + b1_ref[...]
    u = u * jax.nn.sigmoid(u)
    v = jnp.dot(u, w2_ref[...], preferred_element_type=jnp.float32) + b2_ref[...]
    out_ref[...] = v * jax.nn.sigmoid(v)


_node_call = pl.pallas_call(
    _node_body,
    out_shape=jax.ShapeDtypeStruct((N, H), jnp.float32),
)


def _pack_h(h):
    h4 = h.reshape(N, 4, 32)
    lo = h4[:, :, :NLANES].reshape(N, HH)
    hi = h4[:, :, NLANES:].reshape(N, HH)
    hw = _pack_words(lo, hi)
    return hw[:, :HW], hw[:, HW:]


def kernel(z, edge_index, edge_attr, batch, emb, We, be, W1, b1, W2, b2):
    z = z.astype(jnp.int32)
    src = edge_index[0].astype(jnp.int32)
    dst = edge_index[1].astype(jnp.int32)
    ea = edge_attr.astype(jnp.float32)

    z_pad = jnp.concatenate([z, jnp.zeros((NPAD - N,), jnp.int32)])
    pad_n = EPAD - src.shape[0]
    pad_i = jnp.arange(pad_n, dtype=jnp.int32)
    src_p = jnp.concatenate([src, pad_i % N])
    dst_p = jnp.concatenate([dst, N + pad_i % (NPAD - N)])
    dst_p = dst_p.reshape(NS, NG, GRP)
    ea_p = jnp.concatenate([ea, jnp.zeros((EPAD - ea.shape[0], NE), jnp.float32)])
    zeros = jnp.zeros((ZROWS, HH), jnp.float32)

    ea_pairs = ea_p.reshape(EPAD // 2, 2 * NE)
    wh = We.reshape(L, NE, NC, HH).transpose(0, 2, 1, 3)
    wz = jnp.zeros((L, NC, NE, HH), jnp.float32)
    wtop = jnp.concatenate([wh, wz], axis=3)
    wbot = jnp.concatenate([wz, wh], axis=3)
    w2cat = jnp.concatenate([wtop, wbot], axis=2)
    w2cat = w2cat.transpose(2, 0, 1, 3).reshape(2 * NE, L * NC * H)[:, COLPERM]
    bh = be.reshape(L, NC, 1, HH)
    b2cat = jnp.concatenate([bh, bh], axis=3).reshape(1, L * NC * H)[:, COLPERM]
    e_pk = _eproj(ea_pairs, w2cat, b2cat)

    x_pad = _emb_gather(emb, z_pad)
    h = x_pad[:N]
    for l in range(L):
        hb0, hb1 = _pack_h(h)
        aggr2 = _edge_call(l, hb0, hb1, src_p, dst_p, e_pk, zeros)
        h = _node_call(h, aggr2, W1[l], b1[l].reshape(1, H),
                       W2[l], b2[l].reshape(1, H))
    return (h, batch)
